# Initial kernel scaffold; baseline (speedup 1.0000x reference)
import jax
import jax.numpy as jnp
from jax.experimental import pallas as pl


def kernel(x, edge_index, W1, b1, W2, b2, W3, b3, W4, b4, W5, b5, Wl, bl):
    raise NotImplementedError("write your pallas kernel here")


# speedup vs baseline: 11.3593x; 11.3593x over previous
import functools

import jax
import jax.numpy as jnp
from jax import lax
from jax.experimental import pallas as pl
from jax.experimental.pallas import tpu as pltpu
from jax.experimental.pallas import tpu_sc as plsc

N = 10000
E = 320000
NPAD = 10240
NSUB = 16
BM = 1000
RZ = NPAD // NSUB
BE = 200
F32 = jnp.float32


def _sc_mesh():
    return plsc.VectorSubcoreMesh(core_axis_name="c", subcore_axis_name="s")


def _sc_prop_edge(gidx, sidx, table, zeros):
    @functools.partial(
        pl.kernel, mesh=_sc_mesh(),
        out_type=jax.ShapeDtypeStruct((2, NPAD, 128), F32),
        scratch_types=[pltpu.VMEM((BE,), jnp.int32),
                       pltpu.VMEM((BE,), jnp.int32),
                       pltpu.VMEM((BE, 128), F32),
                       pltpu.VMEM_SHARED((NPAD, 128), F32),
                       pltpu.SemaphoreType.DMA],
    )
    def k(g_hbm, s_hbm, table_hbm, zeros_hbm, out_hbm,
          gv, sv, rows, acc, sem):
        core = lax.axis_index("c")
        sub = lax.axis_index("s")
        wid = sub * 2 + core
        pltpu.sync_copy(zeros_hbm, acc.at[pl.ds(sub * RZ, RZ)])
        plsc.subcore_barrier()
        eper = E // 32
        base = wid * eper

        def body(kk, carry):
            off = pl.multiple_of(base + kk * BE, 8)
            pltpu.sync_copy(g_hbm.at[pl.ds(off, BE)], gv)
            pltpu.sync_copy(s_hbm.at[pl.ds(off, BE)], sv)
            pltpu.async_copy(table_hbm.at[gv], rows, sem).wait()
            pltpu.sync_copy(rows, acc.at[sv], add=True)
            return carry

        lax.fori_loop(0, eper // BE, body, 0)
        plsc.subcore_barrier()
        pltpu.sync_copy(acc.at[pl.ds(sub * RZ, RZ)],
                        out_hbm.at[core].at[pl.ds(sub * RZ, RZ)])

    return k(gidx, sidx, table, zeros)


def _sc_prop_chunk(src, dst, table, zeros):
    C = table.shape[0]
    C2 = C // 2

    @functools.partial(
        pl.kernel, mesh=_sc_mesh(),
        out_type=jax.ShapeDtypeStruct((C, NPAD, 128), F32),
        scratch_types=[pltpu.VMEM((BE,), jnp.int32),
                       pltpu.VMEM((BE,), jnp.int32),
                       pltpu.VMEM((BE, 128), F32),
                       pltpu.VMEM_SHARED((NPAD, 128), F32),
                       pltpu.SemaphoreType.DMA],
    )
    def k(src_hbm, dst_hbm, table_hbm, zeros_hbm, out_hbm,
          srcv, dstv, rows, acc, sem):
        core = lax.axis_index("c")
        sub = lax.axis_index("s")
        eper = E // NSUB
        base = sub * eper
        pltpu.sync_copy(zeros_hbm, acc.at[pl.ds(sub * RZ, RZ)])
        plsc.subcore_barrier()
        for ci in range(C2):
            c = core * C2 + ci

            def body(kk, carry):
                off = pl.multiple_of(base + kk * BE, 8)
                pltpu.sync_copy(src_hbm.at[pl.ds(off, BE)], srcv)
                pltpu.sync_copy(dst_hbm.at[pl.ds(off, BE)], dstv)
                pltpu.async_copy(table_hbm.at[c].at[srcv], rows, sem).wait()
                pltpu.sync_copy(rows, acc.at[dstv], add=True)
                return carry

            lax.fori_loop(0, eper // BE, body, 0)
            plsc.subcore_barrier()
            pltpu.sync_copy(acc.at[pl.ds(sub * RZ, RZ)],
                            out_hbm.at[c].at[pl.ds(sub * RZ, RZ)])
            if ci + 1 < C2:
                pltpu.sync_copy(zeros_hbm, acc.at[pl.ds(sub * RZ, RZ)])
            plsc.subcore_barrier()

    return k(src, dst, table, zeros)


def _tc_pre(degt, x):
    def body(deg_ref, x_ref, dinv_ref, dinvb_ref, xp_ref):
        d = deg_ref[0][:, 0:1] + deg_ref[1][:, 0:1] + 1.0
        dinv = lax.rsqrt(d)
        dinv_ref[...] = dinv
        dinvb_ref[...] = jnp.broadcast_to(dinv, dinvb_ref.shape)
        xp_ref[...] = x_ref[...] * dinv

    return pl.pallas_call(
        body,
        grid=(N // BM,),
        in_specs=[pl.BlockSpec((2, BM, 128), lambda i: (0, i, 0)),
                  pl.BlockSpec((BM, 128), lambda i: (i, 0))],
        out_specs=[pl.BlockSpec((BM, 1), lambda i: (i, 0)),
                   pl.BlockSpec((BM, 128), lambda i: (i, 0)),
                   pl.BlockSpec((BM, 128), lambda i: (i, 0))],
        out_shape=[jax.ShapeDtypeStruct((N, 1), F32),
                   jax.ShapeDtypeStruct((N, 128), F32),
                   jax.ShapeDtypeStruct((N, 128), F32)],
    )(degt, x)


def _tc_mm1(t1, xp, dinv, W1, b1r):
    BN = 512

    def body(t_ref, xp_ref, dinv_ref, w_ref, b_ref, o_ref):
        a = dinv_ref[...] * (t_ref[0] + t_ref[1] + xp_ref[...])
        h = jnp.dot(a, w_ref[...], preferred_element_type=F32)
        o_ref[...] = jnp.maximum(h + b_ref[...], 0.0)

    return pl.pallas_call(
        body,
        grid=(N // BM, 4096 // BN),
        in_specs=[pl.BlockSpec((2, BM, 128), lambda i, j: (0, i, 0)),
                  pl.BlockSpec((BM, 128), lambda i, j: (i, 0)),
                  pl.BlockSpec((BM, 1), lambda i, j: (i, 0)),
                  pl.BlockSpec((128, BN), lambda i, j: (0, j)),
                  pl.BlockSpec((1, BN), lambda i, j: (0, j))],
        out_specs=pl.BlockSpec((BM, BN), lambda i, j: (i, j)),
        out_shape=jax.ShapeDtypeStruct((N, 4096), F32),
    )(t1, xp, dinv, W1, b1r)


def _tc_mm2(h1, W2, dinv):
    BK, BN = 512, 512
    KG = 4096 // BK

    def body(h_ref, w_ref, dinv_ref, o_ref, acc):
        kk = pl.program_id(2)

        @pl.when(kk == 0)
        def _():
            acc[...] = jnp.zeros_like(acc)

        acc[...] += jnp.dot(h_ref[...], w_ref[...], preferred_element_type=F32)

        @pl.when(kk == KG - 1)
        def _():
            y = dinv_ref[...] * acc[...]
            for cc in range(BN // 128):
                o_ref[cc] = y[:, cc * 128:(cc + 1) * 128]

    return pl.pallas_call(
        body,
        grid=(N // BM, 1024 // BN, KG),
        in_specs=[pl.BlockSpec((BM, BK), lambda i, j, k: (i, k)),
                  pl.BlockSpec((BK, BN), lambda i, j, k: (k, j)),
                  pl.BlockSpec((BM, 1), lambda i, j, k: (i, 0))],
        out_specs=pl.BlockSpec((BN // 128, BM, 128), lambda i, j, k: (j, i, 0)),
        out_shape=jax.ShapeDtypeStruct((8, N, 128), F32),
        scratch_shapes=[pltpu.VMEM((BM, BN), F32)],
    )(h1, W2, dinv)


def _tc_mm3(t2, y2, b2r, dinv, W3):
    KG = 8

    def body(t_ref, y_ref, b_ref, dinv_ref, w_ref, o_ref, acc):
        kk = pl.program_id(1)

        @pl.when(kk == 0)
        def _():
            acc[...] = jnp.zeros_like(acc)

        a = jnp.maximum(dinv_ref[...] * (t_ref[0] + y_ref[0]) + b_ref[0], 0.0)
        acc[...] += jnp.dot(a, w_ref[...], preferred_element_type=F32)

        @pl.when(kk == KG - 1)
        def _():
            y = dinv_ref[...] * acc[...]
            o_ref[0] = y[:, 0:128]
            o_ref[1] = y[:, 128:256]

    return pl.pallas_call(
        body,
        grid=(N // BM, KG),
        in_specs=[pl.BlockSpec((1, BM, 128), lambda i, k: (k, i, 0)),
                  pl.BlockSpec((1, BM, 128), lambda i, k: (k, i, 0)),
                  pl.BlockSpec((1, 1, 128), lambda i, k: (k, 0, 0)),
                  pl.BlockSpec((BM, 1), lambda i, k: (i, 0)),
                  pl.BlockSpec((128, 256), lambda i, k: (k, 0))],
        out_specs=pl.BlockSpec((2, BM, 128), lambda i, k: (0, i, 0)),
        out_shape=jax.ShapeDtypeStruct((2, N, 128), F32),
        scratch_shapes=[pltpu.VMEM((BM, 256), F32)],
    )(t2, y2, b2r, dinv, W3)


def _tc_mm4(t3, y3, b3r, dinv, W4p):
    KG = 2

    def body(t_ref, y_ref, b_ref, dinv_ref, w_ref, o_ref, acc):
        kk = pl.program_id(1)

        @pl.when(kk == 0)
        def _():
            acc[...] = jnp.zeros_like(acc)

        a = jnp.maximum(dinv_ref[...] * (t_ref[0] + y_ref[0]) + b_ref[0], 0.0)
        acc[...] += jnp.dot(a, w_ref[...], preferred_element_type=F32)

        @pl.when(kk == KG - 1)
        def _():
            o_ref[...] = dinv_ref[...] * acc[...]

    return pl.pallas_call(
        body,
        grid=(N // BM, KG),
        in_specs=[pl.BlockSpec((1, BM, 128), lambda i, k: (k, i, 0)),
                  pl.BlockSpec((1, BM, 128), lambda i, k: (k, i, 0)),
                  pl.BlockSpec((1, 1, 128), lambda i, k: (k, 0, 0)),
                  pl.BlockSpec((BM, 1), lambda i, k: (i, 0)),
                  pl.BlockSpec((128, 128), lambda i, k: (k, 0))],
        out_specs=pl.BlockSpec((BM, 128), lambda i, k: (i, 0)),
        out_shape=jax.ShapeDtypeStruct((N, 128), F32),
        scratch_shapes=[pltpu.VMEM((BM, 128), F32)],
    )(t3, y3, b3r, dinv, W4p)


def _tc_final(t4, y4, dinv, gt, b4p, W5p, b5r, Wl, blr):
    MG = N // BM

    def body(t_ref, y_ref, dinv_ref, g_ref, b4_ref, w5_ref, b5_ref,
             wl_ref, bl_ref, o_ref, acc):
        i = pl.program_id(0)

        @pl.when(i == 0)
        def _():
            acc[...] = jnp.zeros_like(acc)

        dv = dinv_ref[...]
        a5 = jnp.maximum(
            dv * (t_ref[0] + t_ref[1] + y_ref[...]) + b4_ref[...], 0.0)
        s = dv * (g_ref[0][:, 0:1] + g_ref[1][:, 0:1] + dv)
        acc[...] += jnp.sum(s * a5, axis=0, keepdims=True)

        @pl.when(i == MG - 1)
        def _():
            pooled = jnp.dot(acc[...] / float(N), w5_ref[...],
                             preferred_element_type=F32) + b5_ref[...]
            o_ref[...] = jnp.dot(pooled, wl_ref[...],
                                 preferred_element_type=F32) + bl_ref[...]

    return pl.pallas_call(
        body,
        grid=(MG,),
        in_specs=[pl.BlockSpec((2, BM, 128), lambda i: (0, i, 0)),
                  pl.BlockSpec((BM, 128), lambda i: (i, 0)),
                  pl.BlockSpec((BM, 1), lambda i: (i, 0)),
                  pl.BlockSpec((2, BM, 128), lambda i: (0, i, 0)),
                  pl.BlockSpec((1, 128), lambda i: (0, 0)),
                  pl.BlockSpec((128, 32), lambda i: (0, 0)),
                  pl.BlockSpec((1, 32), lambda i: (0, 0)),
                  pl.BlockSpec((32, 3), lambda i: (0, 0)),
                  pl.BlockSpec((1, 3), lambda i: (0, 0))],
        out_specs=pl.BlockSpec((1, 3), lambda i: (0, 0)),
        out_shape=jax.ShapeDtypeStruct((1, 3), F32),
        scratch_shapes=[pltpu.VMEM((1, 128), F32)],
    )(t4, y4, dinv, gt, b4p, W5p, b5r, Wl, blr)


def kernel(x, edge_index, W1, b1, W2, b2, W3, b3, W4, b4, W5, b5, Wl, bl):
    src = edge_index[0]
    dst = edge_index[1]
    z128 = jnp.zeros((RZ, 128), F32)
    ones_t = jnp.ones((N, 128), F32)
    W4p = jnp.concatenate([W4, jnp.zeros((256, 64), F32)], axis=1)
    W5p = jnp.concatenate([W5, jnp.zeros((64, 32), F32)], axis=0)
    b4p = jnp.concatenate([b4, jnp.zeros((64,), F32)]).reshape(1, 128)

    degt = _sc_prop_edge(src, dst, ones_t, z128)
    dinv, dinvb, xp = _tc_pre(degt, x)
    gt = _sc_prop_edge(dst, src, dinvb, z128)

    t1 = _sc_prop_edge(src, dst, xp, z128)
    h1 = _tc_mm1(t1, xp, dinv, W1, b1.reshape(1, -1))
    y2 = _tc_mm2(h1, W2, dinv)
    t2 = _sc_prop_chunk(src, dst, y2, z128)
    y3 = _tc_mm3(t2, y2, b2.reshape(8, 1, 128), dinv, W3)
    t3 = _sc_prop_chunk(src, dst, y3, z128)
    y4 = _tc_mm4(t3, y3, b3.reshape(2, 1, 128), dinv, W4p)
    t4 = _sc_prop_edge(src, dst, y4, z128)
    out = _tc_final(t4, y4, dinv, gt, b4p,
                    W5p, b5.reshape(1, -1), Wl, bl.reshape(1, -1))
    return out


# speedup vs baseline: 12.4425x; 1.0954x over previous
import functools

import jax
import jax.numpy as jnp
from jax import lax
from jax.experimental import pallas as pl
from jax.experimental.pallas import tpu as pltpu
from jax.experimental.pallas import tpu_sc as plsc

N = 10000
E = 320000
NPAD = 10240
NSUB = 16
BM = 1000
RZ = NPAD // NSUB
BE = 80
F32 = jnp.float32


def _sc_mesh():
    return plsc.VectorSubcoreMesh(core_axis_name="c", subcore_axis_name="s")


def _pipelined_blocks(nb, base, g_hbm, s_hbm, table, gvA, svA, rowsA, semA,
                      gvB, svB, rowsB, semB, acc):
    def _idx(b, gv, sv):
        off = pl.multiple_of(base + b * BE, 8)
        pltpu.sync_copy(g_hbm.at[pl.ds(off, BE)], gv)
        pltpu.sync_copy(s_hbm.at[pl.ds(off, BE)], sv)

    _idx(0, gvA, svA)
    pltpu.async_copy(table.at[gvA], rowsA, semA)

    def pair(kk2, carry):
        b0 = kk2 * 2
        b1 = b0 + 1

        @pl.when(b1 < nb)
        def _():
            _idx(b1, gvB, svB)
            pltpu.async_copy(table.at[gvB], rowsB, semB)

        pltpu.make_async_copy(table.at[gvA], rowsA, semA).wait()
        pltpu.sync_copy(rowsA, acc.at[svA], add=True)

        @pl.when(b0 + 2 < nb)
        def _():
            _idx(b0 + 2, gvA, svA)
            pltpu.async_copy(table.at[gvA], rowsA, semA)

        @pl.when(b1 < nb)
        def _():
            pltpu.make_async_copy(table.at[gvB], rowsB, semB).wait()
            pltpu.sync_copy(rowsB, acc.at[svB], add=True)

        return carry

    lax.fori_loop(0, (nb + 1) // 2, pair, 0)


def _sc_prop_edge(gidx, sidx, table, zeros):
    @functools.partial(
        pl.kernel, mesh=_sc_mesh(),
        out_type=jax.ShapeDtypeStruct((2, NPAD, 128), F32),
        scratch_types=[pltpu.VMEM((BE,), jnp.int32),
                       pltpu.VMEM((BE,), jnp.int32),
                       pltpu.VMEM((BE, 128), F32),
                       pltpu.VMEM((BE,), jnp.int32),
                       pltpu.VMEM((BE,), jnp.int32),
                       pltpu.VMEM((BE, 128), F32),
                       pltpu.VMEM_SHARED((NPAD, 128), F32),
                       pltpu.SemaphoreType.DMA,
                       pltpu.SemaphoreType.DMA],
    )
    def k(g_hbm, s_hbm, table_hbm, zeros_hbm, out_hbm,
          gvA, svA, rowsA, gvB, svB, rowsB, acc, semA, semB):
        core = lax.axis_index("c")
        sub = lax.axis_index("s")
        wid = sub * 2 + core
        pltpu.sync_copy(zeros_hbm, acc.at[pl.ds(sub * RZ, RZ)])
        plsc.subcore_barrier()
        eper = E // 32
        _pipelined_blocks(eper // BE, wid * eper, g_hbm, s_hbm, table_hbm,
                          gvA, svA, rowsA, semA, gvB, svB, rowsB, semB, acc)
        plsc.subcore_barrier()
        pltpu.sync_copy(acc.at[pl.ds(sub * RZ, RZ)],
                        out_hbm.at[core].at[pl.ds(sub * RZ, RZ)])

    return k(gidx, sidx, table, zeros)


def _sc_deg_pass(dst, ones_rows, zeros):
    @functools.partial(
        pl.kernel, mesh=_sc_mesh(),
        out_type=jax.ShapeDtypeStruct((2, NPAD, 128), F32),
        scratch_types=[pltpu.VMEM((BE,), jnp.int32),
                       pltpu.VMEM((BE,), jnp.int32),
                       pltpu.VMEM((BE, 128), F32),
                       pltpu.VMEM_SHARED((NPAD, 128), F32)],
    )
    def k(s_hbm, ones_hbm, zeros_hbm, out_hbm, svA, svB, ones_v, acc):
        core = lax.axis_index("c")
        sub = lax.axis_index("s")
        wid = sub * 2 + core
        pltpu.sync_copy(zeros_hbm, acc.at[pl.ds(sub * RZ, RZ)])
        pltpu.sync_copy(ones_hbm, ones_v)
        plsc.subcore_barrier()
        eper = E // 32
        base = wid * eper
        nb = eper // BE

        def pair(kk2, carry):
            b0 = kk2 * 2
            b1 = b0 + 1
            off0 = pl.multiple_of(base + b0 * BE, 8)
            pltpu.sync_copy(s_hbm.at[pl.ds(off0, BE)], svA)

            @pl.when(b1 < nb)
            def _():
                off1 = pl.multiple_of(base + b1 * BE, 8)
                pltpu.sync_copy(s_hbm.at[pl.ds(off1, BE)], svB)

            pltpu.sync_copy(ones_v, acc.at[svA], add=True)

            @pl.when(b1 < nb)
            def _():
                pltpu.sync_copy(ones_v, acc.at[svB], add=True)

            return carry

        lax.fori_loop(0, (nb + 1) // 2, pair, 0)
        plsc.subcore_barrier()
        pltpu.sync_copy(acc.at[pl.ds(sub * RZ, RZ)],
                        out_hbm.at[core].at[pl.ds(sub * RZ, RZ)])

    return k(dst, ones_rows, zeros)


def _sc_prop_chunk(src, dst, table, zeros):
    C = table.shape[0]
    C2 = C // 2

    @functools.partial(
        pl.kernel, mesh=_sc_mesh(),
        out_type=jax.ShapeDtypeStruct((C, NPAD, 128), F32),
        scratch_types=[pltpu.VMEM((BE,), jnp.int32),
                       pltpu.VMEM((BE,), jnp.int32),
                       pltpu.VMEM((BE, 128), F32),
                       pltpu.VMEM((BE,), jnp.int32),
                       pltpu.VMEM((BE,), jnp.int32),
                       pltpu.VMEM((BE, 128), F32),
                       pltpu.VMEM_SHARED((NPAD, 128), F32),
                       pltpu.SemaphoreType.DMA,
                       pltpu.SemaphoreType.DMA],
    )
    def k(src_hbm, dst_hbm, table_hbm, zeros_hbm, out_hbm,
          gvA, svA, rowsA, gvB, svB, rowsB, acc, semA, semB):
        core = lax.axis_index("c")
        sub = lax.axis_index("s")
        eper = E // NSUB
        base = sub * eper
        pltpu.sync_copy(zeros_hbm, acc.at[pl.ds(sub * RZ, RZ)])
        plsc.subcore_barrier()
        for ci in range(C2):
            c = core * C2 + ci
            _pipelined_blocks(eper // BE, base, src_hbm, dst_hbm,
                              table_hbm.at[c], gvA, svA, rowsA, semA,
                              gvB, svB, rowsB, semB, acc)
            plsc.subcore_barrier()
            pltpu.sync_copy(acc.at[pl.ds(sub * RZ, RZ)],
                            out_hbm.at[c].at[pl.ds(sub * RZ, RZ)])
            if ci + 1 < C2:
                pltpu.sync_copy(zeros_hbm, acc.at[pl.ds(sub * RZ, RZ)])
            plsc.subcore_barrier()

    return k(src, dst, table, zeros)


def _tc_pre(degt, x):
    def body(deg_ref, x_ref, dinv_ref, dinvb_ref, xp_ref):
        d = deg_ref[0][:, 0:1] + deg_ref[1][:, 0:1] + 1.0
        dinv = lax.rsqrt(d)
        dinv_ref[...] = dinv
        dinvb_ref[...] = jnp.broadcast_to(dinv, dinvb_ref.shape)
        xp_ref[...] = x_ref[...] * dinv

    return pl.pallas_call(
        body,
        grid=(N // BM,),
        in_specs=[pl.BlockSpec((2, BM, 128), lambda i: (0, i, 0)),
                  pl.BlockSpec((BM, 128), lambda i: (i, 0))],
        out_specs=[pl.BlockSpec((BM, 1), lambda i: (i, 0)),
                   pl.BlockSpec((BM, 128), lambda i: (i, 0)),
                   pl.BlockSpec((BM, 128), lambda i: (i, 0))],
        out_shape=[jax.ShapeDtypeStruct((N, 1), F32),
                   jax.ShapeDtypeStruct((N, 128), F32),
                   jax.ShapeDtypeStruct((N, 128), F32)],
    )(degt, x)


def _tc_mm1(t1, xp, dinv, W1, b1r):
    BN = 512

    def body(t_ref, xp_ref, dinv_ref, w_ref, b_ref, o_ref):
        a = dinv_ref[...] * (t_ref[0] + t_ref[1] + xp_ref[...])
        h = jnp.dot(a, w_ref[...], preferred_element_type=F32)
        o_ref[...] = jnp.maximum(h + b_ref[...], 0.0)

    return pl.pallas_call(
        body,
        grid=(N // BM, 4096 // BN),
        in_specs=[pl.BlockSpec((2, BM, 128), lambda i, j: (0, i, 0)),
                  pl.BlockSpec((BM, 128), lambda i, j: (i, 0)),
                  pl.BlockSpec((BM, 1), lambda i, j: (i, 0)),
                  pl.BlockSpec((128, BN), lambda i, j: (0, j)),
                  pl.BlockSpec((1, BN), lambda i, j: (0, j))],
        out_specs=pl.BlockSpec((BM, BN), lambda i, j: (i, j)),
        out_shape=jax.ShapeDtypeStruct((N, 4096), F32),
    )(t1, xp, dinv, W1, b1r)


def _tc_mm2(h1, W2, dinv):
    BK, BN = 512, 512
    KG = 4096 // BK

    def body(h_ref, w_ref, dinv_ref, o_ref, acc):
        kk = pl.program_id(2)

        @pl.when(kk == 0)
        def _():
            acc[...] = jnp.zeros_like(acc)

        acc[...] += jnp.dot(h_ref[...], w_ref[...], preferred_element_type=F32)

        @pl.when(kk == KG - 1)
        def _():
            y = dinv_ref[...] * acc[...]
            for cc in range(BN // 128):
                o_ref[cc] = y[:, cc * 128:(cc + 1) * 128]

    return pl.pallas_call(
        body,
        grid=(N // BM, 1024 // BN, KG),
        in_specs=[pl.BlockSpec((BM, BK), lambda i, j, k: (i, k)),
                  pl.BlockSpec((BK, BN), lambda i, j, k: (k, j)),
                  pl.BlockSpec((BM, 1), lambda i, j, k: (i, 0))],
        out_specs=pl.BlockSpec((BN // 128, BM, 128), lambda i, j, k: (j, i, 0)),
        out_shape=jax.ShapeDtypeStruct((8, N, 128), F32),
        scratch_shapes=[pltpu.VMEM((BM, BN), F32)],
    )(h1, W2, dinv)


def _tc_mm3(t2, y2, b2r, dinv, W3):
    KG = 8

    def body(t_ref, y_ref, b_ref, dinv_ref, w_ref, o_ref, acc):
        kk = pl.program_id(1)

        @pl.when(kk == 0)
        def _():
            acc[...] = jnp.zeros_like(acc)

        a = jnp.maximum(dinv_ref[...] * (t_ref[0] + y_ref[0]) + b_ref[0], 0.0)
        acc[...] += jnp.dot(a, w_ref[...], preferred_element_type=F32)

        @pl.when(kk == KG - 1)
        def _():
            y = dinv_ref[...] * acc[...]
            o_ref[0] = y[:, 0:128]
            o_ref[1] = y[:, 128:256]

    return pl.pallas_call(
        body,
        grid=(N // BM, KG),
        in_specs=[pl.BlockSpec((1, BM, 128), lambda i, k: (k, i, 0)),
                  pl.BlockSpec((1, BM, 128), lambda i, k: (k, i, 0)),
                  pl.BlockSpec((1, 1, 128), lambda i, k: (k, 0, 0)),
                  pl.BlockSpec((BM, 1), lambda i, k: (i, 0)),
                  pl.BlockSpec((128, 256), lambda i, k: (k, 0))],
        out_specs=pl.BlockSpec((2, BM, 128), lambda i, k: (0, i, 0)),
        out_shape=jax.ShapeDtypeStruct((2, N, 128), F32),
        scratch_shapes=[pltpu.VMEM((BM, 256), F32)],
    )(t2, y2, b2r, dinv, W3)


def _tc_mm4(t3, y3, b3r, dinv, W4p):
    KG = 2

    def body(t_ref, y_ref, b_ref, dinv_ref, w_ref, o_ref, acc):
        kk = pl.program_id(1)

        @pl.when(kk == 0)
        def _():
            acc[...] = jnp.zeros_like(acc)

        a = jnp.maximum(dinv_ref[...] * (t_ref[0] + y_ref[0]) + b_ref[0], 0.0)
        acc[...] += jnp.dot(a, w_ref[...], preferred_element_type=F32)

        @pl.when(kk == KG - 1)
        def _():
            o_ref[...] = dinv_ref[...] * acc[...]

    return pl.pallas_call(
        body,
        grid=(N // BM, KG),
        in_specs=[pl.BlockSpec((1, BM, 128), lambda i, k: (k, i, 0)),
                  pl.BlockSpec((1, BM, 128), lambda i, k: (k, i, 0)),
                  pl.BlockSpec((1, 1, 128), lambda i, k: (k, 0, 0)),
                  pl.BlockSpec((BM, 1), lambda i, k: (i, 0)),
                  pl.BlockSpec((128, 128), lambda i, k: (k, 0))],
        out_specs=pl.BlockSpec((BM, 128), lambda i, k: (i, 0)),
        out_shape=jax.ShapeDtypeStruct((N, 128), F32),
        scratch_shapes=[pltpu.VMEM((BM, 128), F32)],
    )(t3, y3, b3r, dinv, W4p)


def _tc_final(t4, y4, dinv, gt, b4p, W5p, b5r, Wl, blr):
    MG = N // BM

    def body(t_ref, y_ref, dinv_ref, g_ref, b4_ref, w5_ref, b5_ref,
             wl_ref, bl_ref, o_ref, acc):
        i = pl.program_id(0)

        @pl.when(i == 0)
        def _():
            acc[...] = jnp.zeros_like(acc)

        dv = dinv_ref[...]
        a5 = jnp.maximum(
            dv * (t_ref[0] + t_ref[1] + y_ref[...]) + b4_ref[...], 0.0)
        s = dv * (g_ref[0][:, 0:1] + g_ref[1][:, 0:1] + dv)
        acc[...] += jnp.sum(s * a5, axis=0, keepdims=True)

        @pl.when(i == MG - 1)
        def _():
            pooled = jnp.dot(acc[...] / float(N), w5_ref[...],
                             preferred_element_type=F32) + b5_ref[...]
            o_ref[...] = jnp.dot(pooled, wl_ref[...],
                                 preferred_element_type=F32) + bl_ref[...]

    return pl.pallas_call(
        body,
        grid=(MG,),
        in_specs=[pl.BlockSpec((2, BM, 128), lambda i: (0, i, 0)),
                  pl.BlockSpec((BM, 128), lambda i: (i, 0)),
                  pl.BlockSpec((BM, 1), lambda i: (i, 0)),
                  pl.BlockSpec((2, BM, 128), lambda i: (0, i, 0)),
                  pl.BlockSpec((1, 128), lambda i: (0, 0)),
                  pl.BlockSpec((128, 32), lambda i: (0, 0)),
                  pl.BlockSpec((1, 32), lambda i: (0, 0)),
                  pl.BlockSpec((32, 3), lambda i: (0, 0)),
                  pl.BlockSpec((1, 3), lambda i: (0, 0))],
        out_specs=pl.BlockSpec((1, 3), lambda i: (0, 0)),
        out_shape=jax.ShapeDtypeStruct((1, 3), F32),
        scratch_shapes=[pltpu.VMEM((1, 128), F32)],
    )(t4, y4, dinv, gt, b4p, W5p, b5r, Wl, blr)


def kernel(x, edge_index, W1, b1, W2, b2, W3, b3, W4, b4, W5, b5, Wl, bl):
    src = edge_index[0]
    dst = edge_index[1]
    z128 = jnp.zeros((RZ, 128), F32)
    ones_be = jnp.ones((BE, 128), F32)
    W4p = jnp.concatenate([W4, jnp.zeros((256, 64), F32)], axis=1)
    W5p = jnp.concatenate([W5, jnp.zeros((64, 32), F32)], axis=0)
    b4p = jnp.concatenate([b4, jnp.zeros((64,), F32)]).reshape(1, 128)

    degt = _sc_deg_pass(dst, ones_be, z128)
    dinv, dinvb, xp = _tc_pre(degt, x)
    gt = _sc_prop_edge(dst, src, dinvb, z128)

    t1 = _sc_prop_edge(src, dst, xp, z128)
    h1 = _tc_mm1(t1, xp, dinv, W1, b1.reshape(1, -1))
    y2 = _tc_mm2(h1, W2, dinv)
    t2 = _sc_prop_chunk(src, dst, y2, z128)
    y3 = _tc_mm3(t2, y2, b2.reshape(8, 1, 128), dinv, W3)
    t3 = _sc_prop_chunk(src, dst, y3, z128)
    y4 = _tc_mm4(t3, y3, b3.reshape(2, 1, 128), dinv, W4p)
    t4 = _sc_prop_edge(src, dst, y4, z128)
    out = _tc_final(t4, y4, dinv, gt, b4p,
                    W5p, b5.reshape(1, -1), Wl, bl.reshape(1, -1))
    return out
